# baseline (device time: 208664 ns/iter reference)
import jax
import jax.numpy as jnp
from jax import lax
from jax.experimental import pallas as pl
from jax.experimental.pallas import tpu as pltpu

N_DEV = 16


def kernel(x, router_W, route_idx, expert_W, shared_W):
    n_tok, d = x.shape
    e_per, _, h = expert_W.shape
    n_exp = router_W.shape[1]

    def body(x_ref, rw_ref, idx_ref, ew_ref, sw_ref, out_ref,
             wall_ref, send_sems, recv_sems):
        my = lax.axis_index("i")
        left = lax.rem(my - 1 + N_DEV, N_DEV)
        right = lax.rem(my + 1, N_DEV)

        barrier_sem = pltpu.get_barrier_semaphore()
        pl.semaphore_signal(barrier_sem, inc=1, device_id=(left,),
                            device_id_type=pl.DeviceIdType.MESH)
        pl.semaphore_signal(barrier_sem, inc=1, device_id=(right,),
                            device_id_type=pl.DeviceIdType.MESH)
        pl.semaphore_wait(barrier_sem, 2)

        xf = x_ref[...]
        xb = xf.astype(jnp.bfloat16)
        idx = idx_ref[...]

        scores = jnp.dot(xf, rw_ref[...], preferred_element_type=jnp.float32)
        m = jnp.max(scores, axis=1, keepdims=True)
        p = jnp.exp(scores - m)
        probs = p / jnp.sum(p, axis=1, keepdims=True)
        eids = lax.broadcasted_iota(jnp.int32, (n_tok, n_exp), 1)
        onehot = (eids == idx).astype(jnp.float32)
        prob_tok = jnp.sum(probs * onehot, axis=1, keepdims=True)

        acc = jnp.dot(xb, sw_ref[...].astype(jnp.bfloat16),
                      preferred_element_type=jnp.float32)

        wall_ref[0] = ew_ref[...].astype(jnp.bfloat16)

        for hop in range(N_DEV):
            if hop < N_DEV - 1:
                rdma = pltpu.make_async_remote_copy(
                    src_ref=wall_ref.at[hop],
                    dst_ref=wall_ref.at[hop + 1],
                    send_sem=send_sems.at[hop],
                    recv_sem=recv_sems.at[hop],
                    device_id=(right,),
                    device_id_type=pl.DeviceIdType.MESH,
                )
                rdma.start()

            src = lax.rem(my - hop + N_DEV, N_DEV)
            for j in range(e_per):
                y = jnp.dot(xb, wall_ref[hop, j],
                            preferred_element_type=jnp.float32)
                coef = jnp.where(idx == (src * e_per + j), prob_tok, 0.0)
                acc = acc + coef * y

            if hop < N_DEV - 1:
                rdma.wait()

        out_ref[...] = acc

    return pl.pallas_call(
        body,
        out_shape=jax.ShapeDtypeStruct((n_tok, h), jnp.float32),
        in_specs=[pl.BlockSpec(memory_space=pltpu.VMEM)] * 5,
        out_specs=pl.BlockSpec(memory_space=pltpu.VMEM),
        scratch_shapes=[
            pltpu.VMEM((N_DEV, e_per, d, h), jnp.bfloat16),
            pltpu.SemaphoreType.DMA((N_DEV - 1,)),
            pltpu.SemaphoreType.DMA((N_DEV - 1,)),
        ],
        compiler_params=pltpu.CompilerParams(collective_id=0),
    )(x, router_W, route_idx, expert_W, shared_W)


# device time: 129388 ns/iter; 1.6127x vs baseline; 1.6127x over previous
import jax
import jax.numpy as jnp
from jax import lax
from jax.experimental import pallas as pl
from jax.experimental.pallas import tpu as pltpu

N_DEV = 16
H_R = N_DEV // 2
H_L = N_DEV - 1 - H_R


def kernel(x, router_W, route_idx, expert_W, shared_W):
    n_tok, d = x.shape
    e_per, _, h = expert_W.shape
    n_exp = router_W.shape[1]

    def body(x_ref, rw_ref, idx_ref, ew_ref, sw_ref, out_ref,
             wr_ref, wl_ref, send_r, recv_r, send_l, recv_l):
        my = lax.axis_index("i")
        left = lax.rem(my - 1 + N_DEV, N_DEV)
        right = lax.rem(my + 1, N_DEV)

        barrier_sem = pltpu.get_barrier_semaphore()
        pl.semaphore_signal(barrier_sem, inc=1, device_id=(left,),
                            device_id_type=pl.DeviceIdType.MESH)
        pl.semaphore_signal(barrier_sem, inc=1, device_id=(right,),
                            device_id_type=pl.DeviceIdType.MESH)
        pl.semaphore_wait(barrier_sem, 2)

        xf = x_ref[...]
        xb = xf.astype(jnp.bfloat16)
        idx = idx_ref[...]

        scores = jnp.dot(xf, rw_ref[...], preferred_element_type=jnp.float32)
        m = jnp.max(scores, axis=1, keepdims=True)
        p = jnp.exp(scores - m)
        probs = p / jnp.sum(p, axis=1, keepdims=True)
        eids = lax.broadcasted_iota(jnp.int32, (n_tok, n_exp), 1)
        onehot = (eids == idx).astype(jnp.float32)
        prob_tok = jnp.sum(probs * onehot, axis=1, keepdims=True)

        acc = jnp.dot(xb, sw_ref[...].astype(jnp.bfloat16),
                      preferred_element_type=jnp.float32)

        own = ew_ref[...].astype(jnp.bfloat16)
        wr_ref[0] = own
        wl_ref[0] = own

        def process(w_ref_at_slot, src):
            nonlocal acc
            for j in range(e_per):
                y = jnp.dot(xb, w_ref_at_slot[j],
                            preferred_element_type=jnp.float32)
                coef = jnp.where(idx == (src * e_per + j), prob_tok, 0.0)
                acc = acc + coef * y

        for hop in range(1, H_R + 1):
            rdma_r = pltpu.make_async_remote_copy(
                src_ref=wr_ref.at[hop - 1],
                dst_ref=wr_ref.at[hop],
                send_sem=send_r.at[hop - 1],
                recv_sem=recv_r.at[hop - 1],
                device_id=(right,),
                device_id_type=pl.DeviceIdType.MESH,
            )
            rdma_r.start()
            if hop <= H_L:
                rdma_l = pltpu.make_async_remote_copy(
                    src_ref=wl_ref.at[hop - 1],
                    dst_ref=wl_ref.at[hop],
                    send_sem=send_l.at[hop - 1],
                    recv_sem=recv_l.at[hop - 1],
                    device_id=(left,),
                    device_id_type=pl.DeviceIdType.MESH,
                )
                rdma_l.start()

            process(wr_ref[hop - 1], lax.rem(my - (hop - 1) + N_DEV, N_DEV))
            if hop - 1 >= 1 and hop - 1 <= H_L:
                process(wl_ref[hop - 1], lax.rem(my + (hop - 1), N_DEV))

            rdma_r.wait()
            if hop <= H_L:
                rdma_l.wait()

        process(wr_ref[H_R], lax.rem(my - H_R + N_DEV, N_DEV))

        out_ref[...] = acc

    return pl.pallas_call(
        body,
        out_shape=jax.ShapeDtypeStruct((n_tok, h), jnp.float32),
        in_specs=[pl.BlockSpec(memory_space=pltpu.VMEM)] * 5,
        out_specs=pl.BlockSpec(memory_space=pltpu.VMEM),
        scratch_shapes=[
            pltpu.VMEM((H_R + 1, e_per, d, h), jnp.bfloat16),
            pltpu.VMEM((H_L + 1, e_per, d, h), jnp.bfloat16),
            pltpu.SemaphoreType.DMA((H_R,)),
            pltpu.SemaphoreType.DMA((H_R,)),
            pltpu.SemaphoreType.DMA((H_L,)),
            pltpu.SemaphoreType.DMA((H_L,)),
        ],
        compiler_params=pltpu.CompilerParams(collective_id=0),
    )(x, router_W, route_idx, expert_W, shared_W)


# device time: 85040 ns/iter; 2.4537x vs baseline; 1.5215x over previous
import jax
import jax.numpy as jnp
from jax import lax
from jax.experimental import pallas as pl
from jax.experimental.pallas import tpu as pltpu

N_DEV = 16
CAP = 64


def kernel(x, router_W, route_idx, expert_W, shared_W):
    n_tok, d = x.shape
    e_per, _, h = expert_W.shape
    n_exp = router_W.shape[1]
    n_slot = n_exp * CAP

    def body(x_ref, rw_ref, idx_ref, ew_ref, sw_ref, out_ref,
             disp_ref, recv_ref, ret_ref, recv2_ref,
             s1, r1, s2, r2):
        my = lax.axis_index("i")

        barrier_sem = pltpu.get_barrier_semaphore()
        for o in range(1, N_DEV):
            peer = lax.rem(my + o, N_DEV)
            pl.semaphore_signal(barrier_sem, inc=1, device_id=(peer,),
                                device_id_type=pl.DeviceIdType.MESH)
        pl.semaphore_wait(barrier_sem, N_DEV - 1)

        xf = x_ref[...]
        xb = xf.astype(jnp.bfloat16)
        idx = idx_ref[...]

        scores = jnp.dot(xf, rw_ref[...], preferred_element_type=jnp.float32)
        m = jnp.max(scores, axis=1, keepdims=True)
        p = jnp.exp(scores - m)
        probs = p / jnp.sum(p, axis=1, keepdims=True)
        eids = lax.broadcasted_iota(jnp.int32, (n_tok, n_exp), 1)
        onehot = (eids == idx).astype(jnp.float32)
        prob_tok = jnp.sum(probs * onehot, axis=1, keepdims=True)

        rows = lax.broadcasted_iota(jnp.int32, (n_tok, n_tok), 0)
        cols = lax.broadcasted_iota(jnp.int32, (n_tok, n_tok), 1)
        l_strict = (rows > cols).astype(jnp.float32)
        cum = jnp.dot(l_strict, onehot, preferred_element_type=jnp.float32)
        pos = jnp.sum(onehot * cum, axis=1, keepdims=True).astype(jnp.int32)
        q = jnp.where(pos < CAP, idx * CAP + pos, -1)
        slot_ids = lax.broadcasted_iota(jnp.int32, (n_tok, n_slot), 1)
        p_all = (slot_ids == q).astype(jnp.bfloat16)

        d_all = lax.dot_general(p_all, xb, (((0,), (0,)), ((), ())),
                                preferred_element_type=jnp.float32)
        disp_ref[...] = d_all.astype(jnp.bfloat16).reshape(
            N_DEV, e_per, CAP, d)

        disp_rdmas = []
        for o in range(1, N_DEV):
            t = lax.rem(my + o, N_DEV)
            rdma = pltpu.make_async_remote_copy(
                src_ref=disp_ref.at[t],
                dst_ref=recv_ref.at[my],
                send_sem=s1.at[o],
                recv_sem=r1.at[o],
                device_id=(t,),
                device_id_type=pl.DeviceIdType.MESH,
            )
            rdma.start()
            disp_rdmas.append(rdma)
        recv_ref[my] = disp_ref[my]

        acc = jnp.dot(xb, sw_ref[...].astype(jnp.bfloat16),
                      preferred_element_type=jnp.float32)

        for rdma in disp_rdmas:
            rdma.wait_recv()

        ewb = ew_ref[...].astype(jnp.bfloat16)
        rv = recv_ref[...]
        for j in range(e_per):
            rows = rv[:, j].reshape(N_DEV * CAP, d)
            yj = jnp.dot(rows, ewb[j], preferred_element_type=jnp.float32)
            ret_ref[:, j] = yj.astype(jnp.bfloat16).reshape(N_DEV, CAP, h)

        ret_rdmas = []
        for o in range(1, N_DEV):
            t = lax.rem(my + o, N_DEV)
            rdma = pltpu.make_async_remote_copy(
                src_ref=ret_ref.at[t],
                dst_ref=recv2_ref.at[my],
                send_sem=s2.at[o],
                recv_sem=r2.at[o],
                device_id=(t,),
                device_id_type=pl.DeviceIdType.MESH,
            )
            rdma.start()
            ret_rdmas.append(rdma)
        recv2_ref[my] = ret_ref[my]

        for rdma in ret_rdmas:
            rdma.wait_recv()

        y_flat = recv2_ref[...].reshape(n_slot, h)
        routed = jnp.dot(p_all, y_flat, preferred_element_type=jnp.float32)
        out_ref[...] = acc + prob_tok * routed

        for rdma in disp_rdmas:
            rdma.wait_send()
        for rdma in ret_rdmas:
            rdma.wait_send()

    return pl.pallas_call(
        body,
        out_shape=jax.ShapeDtypeStruct((n_tok, h), jnp.float32),
        in_specs=[pl.BlockSpec(memory_space=pltpu.VMEM)] * 5,
        out_specs=pl.BlockSpec(memory_space=pltpu.VMEM),
        scratch_shapes=[
            pltpu.VMEM((N_DEV, e_per, CAP, d), jnp.bfloat16),
            pltpu.VMEM((N_DEV, e_per, CAP, d), jnp.bfloat16),
            pltpu.VMEM((N_DEV, e_per, CAP, h), jnp.bfloat16),
            pltpu.VMEM((N_DEV, e_per, CAP, h), jnp.bfloat16),
            pltpu.SemaphoreType.DMA((N_DEV,)),
            pltpu.SemaphoreType.DMA((N_DEV,)),
            pltpu.SemaphoreType.DMA((N_DEV,)),
            pltpu.SemaphoreType.DMA((N_DEV,)),
        ],
        compiler_params=pltpu.CompilerParams(collective_id=0),
    )(x, router_W, route_idx, expert_W, shared_W)


# device time: 57003 ns/iter; 3.6606x vs baseline; 1.4919x over previous
import jax
import jax.numpy as jnp
from jax import lax
from jax.experimental import pallas as pl
from jax.experimental.pallas import tpu as pltpu

N_DEV = 16
CAP = 128
META = 128


def kernel(x, router_W, route_idx, expert_W, shared_W):
    n_tok, d = x.shape
    e_per, _, h = expert_W.shape
    n_exp = router_W.shape[1]
    n_slot = N_DEV * CAP
    d_aug = d + META

    def body(x_ref, rw_ref, idx_ref, ew_ref, sw_ref, out_ref,
             disp_ref, recv_ref, ret_ref, recv2_ref,
             s1, r1, s2, r2):
        my = lax.axis_index("i")

        barrier_sem = pltpu.get_barrier_semaphore()
        for o in range(1, N_DEV):
            peer = lax.rem(my + o, N_DEV)
            pl.semaphore_signal(barrier_sem, inc=1, device_id=(peer,),
                                device_id_type=pl.DeviceIdType.MESH)
        pl.semaphore_wait(barrier_sem, N_DEV - 1)

        xf = x_ref[...]
        xb = xf.astype(jnp.bfloat16)
        idx = idx_ref[...]

        scores = jnp.dot(xf, rw_ref[...], preferred_element_type=jnp.float32)
        mx = jnp.max(scores, axis=1, keepdims=True)
        p = jnp.exp(scores - mx)
        probs = p / jnp.sum(p, axis=1, keepdims=True)
        eids = lax.broadcasted_iota(jnp.int32, (n_tok, n_exp), 1)
        onehot = (eids == idx).astype(jnp.float32)
        prob_tok = jnp.sum(probs * onehot, axis=1, keepdims=True)

        dest = idx // e_per
        jloc = lax.rem(idx, e_per)
        dids = lax.broadcasted_iota(jnp.int32, (n_tok, N_DEV), 1)
        doh = (dids == dest).astype(jnp.float32)
        rows = lax.broadcasted_iota(jnp.int32, (n_tok, n_tok), 0)
        cols = lax.broadcasted_iota(jnp.int32, (n_tok, n_tok), 1)
        l_strict = (rows > cols).astype(jnp.float32)
        cum = jnp.dot(l_strict, doh, preferred_element_type=jnp.float32)
        pos = jnp.sum(doh * cum, axis=1, keepdims=True).astype(jnp.int32)
        q = jnp.where(pos < CAP, dest * CAP + pos, -1)
        slot_ids = lax.broadcasted_iota(jnp.int32, (n_tok, n_slot), 1)
        p_all = (slot_ids == q).astype(jnp.bfloat16)

        mids = lax.broadcasted_iota(jnp.int32, (n_tok, META), 1)
        moh = (mids == jloc).astype(jnp.bfloat16)
        xaug = jnp.concatenate([xb, moh], axis=1)

        d_all = lax.dot_general(p_all, xaug, (((0,), (0,)), ((), ())),
                                preferred_element_type=jnp.float32)
        disp_ref[...] = d_all.astype(jnp.bfloat16).reshape(N_DEV, CAP, d_aug)

        disp_rdmas = []
        for o in range(1, N_DEV):
            t = lax.rem(my + o, N_DEV)
            rdma = pltpu.make_async_remote_copy(
                src_ref=disp_ref.at[t],
                dst_ref=recv_ref.at[my],
                send_sem=s1.at[o],
                recv_sem=r1.at[o],
                device_id=(t,),
                device_id_type=pl.DeviceIdType.MESH,
            )
            rdma.start()
            disp_rdmas.append(rdma)
        recv_ref[my] = disp_ref[my]

        acc = jnp.dot(xb, sw_ref[...].astype(jnp.bfloat16),
                      preferred_element_type=jnp.float32)

        for rdma in disp_rdmas:
            rdma.wait_recv()

        ewb = ew_ref[...].astype(jnp.bfloat16)
        rv = recv_ref[...].reshape(N_DEV * CAP, d_aug)
        xpart = rv[:, :d]
        meta = rv[:, d:].astype(jnp.float32)
        ysel = jnp.zeros((N_DEV * CAP, h), jnp.float32)
        for j in range(e_per):
            yj = jnp.dot(xpart, ewb[j], preferred_element_type=jnp.float32)
            ysel = ysel + meta[:, j:j + 1] * yj
        ret_ref[...] = ysel.astype(jnp.bfloat16).reshape(N_DEV, CAP, h)

        ret_rdmas = []
        for o in range(1, N_DEV):
            t = lax.rem(my + o, N_DEV)
            rdma = pltpu.make_async_remote_copy(
                src_ref=ret_ref.at[t],
                dst_ref=recv2_ref.at[my],
                send_sem=s2.at[o],
                recv_sem=r2.at[o],
                device_id=(t,),
                device_id_type=pl.DeviceIdType.MESH,
            )
            rdma.start()
            ret_rdmas.append(rdma)
        recv2_ref[my] = ret_ref[my]

        for rdma in ret_rdmas:
            rdma.wait_recv()

        y_flat = recv2_ref[...].reshape(n_slot, h)
        routed = jnp.dot(p_all, y_flat, preferred_element_type=jnp.float32)
        out_ref[...] = acc + prob_tok * routed

        for rdma in disp_rdmas:
            rdma.wait_send()
        for rdma in ret_rdmas:
            rdma.wait_send()

    return pl.pallas_call(
        body,
        out_shape=jax.ShapeDtypeStruct((n_tok, h), jnp.float32),
        in_specs=[pl.BlockSpec(memory_space=pltpu.VMEM)] * 5,
        out_specs=pl.BlockSpec(memory_space=pltpu.VMEM),
        scratch_shapes=[
            pltpu.VMEM((N_DEV, CAP, d_aug), jnp.bfloat16),
            pltpu.VMEM((N_DEV, CAP, d_aug), jnp.bfloat16),
            pltpu.VMEM((N_DEV, CAP, h), jnp.bfloat16),
            pltpu.VMEM((N_DEV, CAP, h), jnp.bfloat16),
            pltpu.SemaphoreType.DMA((N_DEV,)),
            pltpu.SemaphoreType.DMA((N_DEV,)),
            pltpu.SemaphoreType.DMA((N_DEV,)),
            pltpu.SemaphoreType.DMA((N_DEV,)),
        ],
        compiler_params=pltpu.CompilerParams(collective_id=0),
    )(x, router_W, route_idx, expert_W, shared_W)
